# baseline (speedup 1.0000x reference)
import functools

import jax
import jax.numpy as jnp
from jax import lax
from jax.experimental import pallas as pl
from jax.experimental.pallas import tpu as pltpu
from jax.experimental.pallas import tpu_sc as plsc

_B = 1024
_I = 100000
_TP1 = 50
_NN = 3
_NPR = _TP1 + _TP1 * _NN
_TOTAL = _B * _NPR
_NC, _NS = 2, 16
_NW = _NC * _NS
_PER_W = _TOTAL // _NW
_NVEC = _PER_W // 16
_WP = 1.0 / (_B * _TP1)
_WN = 1.0 / (_B * _TP1 * _NN)


def _log1p_series(u):
    z = u / (2.0 + u)
    z2 = z * z
    p = 1.0 / 13.0
    for c in (11.0, 9.0, 7.0, 5.0, 3.0):
        p = p * z2 + 1.0 / c
    p = p * z2 + 1.0
    return 2.0 * z * p


def _loss_body(yh_hbm, idx_hbm, out_hbm, idx_v, vals_v, acc_v, sem):
    wid = lax.axis_index("s") * _NC + lax.axis_index("c")
    base = wid * _PER_W
    pltpu.sync_copy(idx_hbm.at[pl.ds(base, _PER_W)], idx_v)
    pltpu.async_copy(yh_hbm.at[idx_v], vals_v, sem).wait()

    lane = lax.iota(jnp.int32, 16)

    def _acc(j, acc):
        x = vals_v[pl.ds(j * 16, 16)]
        pos = ((base + j * 16 + lane) % _NPR) < _TP1
        sgn = jnp.where(pos, -1.0, 1.0)
        w = jnp.where(pos, _WP, _WN)
        t = _log1p_series(jnp.exp(-jnp.abs(x)))
        elem = jnp.maximum(sgn * x, 0.0) + t
        return acc + w * elem

    acc = lax.fori_loop(0, _NVEC, _acc, jnp.zeros((16,), jnp.float32),
                        unroll=False)
    acc_v[...] = acc
    pltpu.sync_copy(acc_v, out_hbm.at[pl.ds(wid * 16, 16)])


@functools.cache
def _sc_loss():
    return pl.kernel(
        _loss_body,
        out_type=jax.ShapeDtypeStruct((_NW * 16,), jnp.float32),
        mesh=plsc.VectorSubcoreMesh(core_axis_name="c", subcore_axis_name="s",
                                    num_cores=_NC, num_subcores=_NS),
        scratch_types=[
            pltpu.VMEM((_PER_W,), jnp.int32),
            pltpu.VMEM((_PER_W,), jnp.float32),
            pltpu.VMEM((16,), jnp.float32),
            pltpu.SemaphoreType.DMA,
        ],
    )


def kernel(y_hat, y_pos, y_neg):
    idx = jnp.concatenate([y_pos, y_neg.reshape(_B, _TP1 * _NN)], axis=1)
    gidx = (idx + jnp.arange(_B, dtype=jnp.int32)[:, None] * _I).reshape(-1)
    flat = jnp.concatenate(
        [y_hat[: _B // 2].reshape(-1), y_hat[_B // 2:].reshape(-1)])
    partials = _sc_loss()(flat, gidx)
    return jnp.sum(partials)


# speedup vs baseline: 1.7554x; 1.7554x over previous
import functools

import jax
import jax.numpy as jnp
from jax import lax
from jax.experimental import pallas as pl
from jax.experimental.pallas import tpu as pltpu
from jax.experimental.pallas import tpu_sc as plsc

_B = 1024
_I = 100000
_TP1 = 50
_NN = 3
_NPR = _TP1 + _TP1 * _NN
_TOTAL = _B * _NPR
_NC, _NS = 2, 16
_NW = _NC * _NS
_PER_W = _TOTAL // _NW
_NVEC = _PER_W // 16
_WP = 1.0 / (_B * _TP1)
_WN = 1.0 / (_B * _TP1 * _NN)


def _log1p_series(u):
    z = u / (2.0 + u)
    z2 = z * z
    p = 1.0 / 13.0
    for c in (11.0, 9.0, 7.0, 5.0, 3.0):
        p = p * z2 + 1.0 / c
    p = p * z2 + 1.0
    return 2.0 * z * p


def _loss_body(yh_hbm, idx_hbm, out_hbm, idx_v, vals_v, acc_v, sem):
    wid = lax.axis_index("s") * _NC + lax.axis_index("c")
    base = wid * _PER_W
    pltpu.sync_copy(idx_hbm.at[pl.ds(base, _PER_W)], idx_v)
    pltpu.async_copy(yh_hbm.at[idx_v], vals_v, sem).wait()

    lane = lax.iota(jnp.int32, 16)

    def _acc(j, acc):
        x = vals_v[pl.ds(j * 16, 16)]
        pos = ((base + j * 16 + lane) % _NPR) < _TP1
        sgn = jnp.where(pos, -1.0, 1.0)
        w = jnp.where(pos, _WP, _WN)
        t = _log1p_series(jnp.exp(-jnp.abs(x)))
        elem = jnp.maximum(sgn * x, 0.0) + t
        return acc + w * elem

    acc = lax.fori_loop(0, _NVEC, _acc, jnp.zeros((16,), jnp.float32),
                        unroll=False)
    acc_v[...] = acc
    pltpu.sync_copy(acc_v, out_hbm.at[pl.ds(wid * 16, 16)])


@functools.cache
def _sc_loss():
    return pl.kernel(
        _loss_body,
        out_type=jax.ShapeDtypeStruct((_NW * 16,), jnp.float32),
        mesh=plsc.VectorSubcoreMesh(core_axis_name="c", subcore_axis_name="s",
                                    num_cores=_NC, num_subcores=_NS),
        scratch_types=[
            pltpu.VMEM((_PER_W,), jnp.int32),
            pltpu.VMEM((_PER_W,), jnp.float32),
            pltpu.VMEM((16,), jnp.float32),
            pltpu.SemaphoreType.DMA,
        ],
    )


def kernel(y_hat, y_pos, y_neg):
    idx = jnp.concatenate([y_pos, y_neg.reshape(_B, _TP1 * _NN)], axis=1)
    gidx = (idx + jnp.arange(_B, dtype=jnp.int32)[:, None] * 100096).reshape(-1)
    flat = jnp.pad(y_hat, ((0, 0), (0, 96))).reshape(-1)
    partials = _sc_loss()(flat, gidx)
    return jnp.sum(partials)
